# baseline (speedup 1.0000x reference)
import functools

import jax
import jax.numpy as jnp
from jax import lax
from jax.experimental import pallas as pl
from jax.experimental.pallas import tpu as pltpu
from jax.experimental.pallas import tpu_sc as plsc

_E = 8
_K = 2
_TM = 128
_TMS = 7
_NEG = -1e30

_NC = 2
_NS = 16
_NW = _NC * _NS


def _silu(v):
    return v * jax.nn.sigmoid(v)


def _router_body(x_ref, rw_ref, ei_ref, g_ref):
    x = x_ref[...]
    rw = rw_ref[...]
    logits = lax.dot_general(rw, x, (((0,), (1,)), ((), ())),
                             preferred_element_type=jnp.float32)
    row = lax.broadcasted_iota(jnp.int32, logits.shape, 0)
    m1 = jnp.max(logits, axis=0, keepdims=True)
    i1 = jnp.min(jnp.where(logits == m1, row, _E), axis=0, keepdims=True)
    l2 = jnp.where(row == i1, _NEG, logits)
    m2 = jnp.max(l2, axis=0, keepdims=True)
    i2 = jnp.min(jnp.where(l2 == m2, row, _E), axis=0, keepdims=True)
    g1 = jax.nn.sigmoid(m1 - m2)
    T = x.shape[0]
    ei_ref[pl.ds(0, T)] = i1[0]
    ei_ref[pl.ds(T, T)] = i2[0]
    g_ref[pl.ds(0, T)] = g1[0]
    g_ref[pl.ds(T, T)] = 1.0 - g1[0]


def _router(x, router_w):
    T, D = x.shape
    return pl.pallas_call(
        _router_body,
        in_specs=[pl.BlockSpec((T, D), lambda: (0, 0)),
                  pl.BlockSpec((D, _E), lambda: (0, 0))],
        out_specs=[pl.BlockSpec((_K * T,), lambda: (0,)),
                   pl.BlockSpec((_K * T,), lambda: (0,))],
        out_shape=[jax.ShapeDtypeStruct((_K * T,), jnp.int32),
                   jax.ShapeDtypeStruct((_K * T,), jnp.float32)],
    )(x, router_w)


def _sc_mesh():
    return plsc.VectorSubcoreMesh(core_axis_name="c", subcore_axis_name="s")


def _splat(v, j):
    return jnp.take(v, jnp.zeros((16,), jnp.int32) + j)


def _prefix(v):
    lane = lax.iota(jnp.int32, 16)
    for st in (1, 2, 4, 8):
        sh = jnp.take(v, jnp.maximum(lane - st, 0))
        v = v + jnp.where(lane >= st, sh, 0)
    return v


def _permute_call(ei, x, NB_pad):
    T, D = x.shape
    A = _K * T
    tpw = T // _NW
    P = A + _E * _TM
    NV = A // 16

    @functools.partial(
        pl.kernel, mesh=_sc_mesh(),
        out_type=[jax.ShapeDtypeStruct((A,), jnp.int32),
                  jax.ShapeDtypeStruct((NB_pad,), jnp.int32),
                  jax.ShapeDtypeStruct((P, D), jnp.float32)],
        scratch_types=[pltpu.VMEM((A,), jnp.int32),
                       pltpu.VMEM((tpw,), jnp.int32),
                       pltpu.VMEM((tpw,), jnp.int32),
                       pltpu.VMEM((tpw, D), jnp.float32),
                       pltpu.VMEM((NB_pad,), jnp.int32),
                       pltpu.SemaphoreType.DMA],
    )
    def permute(ei_hbm, x_hbm, pos_hbm, be_hbm, xp_hbm,
                e_v, pos0_v, pos1_v, rows_v, be_v, sem):
        w = lax.axis_index("s") * _NC + lax.axis_index("c")
        t0 = w * tpw
        pltpu.sync_copy(ei_hbm, e_v)
        cpx = pltpu.async_copy(x_hbm.at[pl.ds(t0, tpw), :], rows_v, sem)
        wv4 = jnp.zeros((16,), jnp.int32) + w * (tpw // 16)

        def scan(u, carry):
            ev = e_v[pl.ds(u * 16, 16)]
            um = lax.rem(u, NV // _K)
            fac = jnp.clip(wv4 - um, 0, 1)
            out = []
            for ex in range(_E):
                m = jnp.where(ev == ex, 1, 0)
                out.append(carry[2 * ex] + m)
                out.append(carry[2 * ex + 1] + m * fac)
            return tuple(out)

        zero = jnp.zeros((16,), jnp.int32)
        acc = lax.fori_loop(0, NV, scan, (zero,) * (2 * _E))

        run_pad = jnp.zeros((16,), jnp.int32)
        base, end = [], []
        for ex in range(_E):
            tot = _splat(_prefix(acc[2 * ex]), 15)
            pri = _splat(_prefix(acc[2 * ex + 1]), 15)
            pad = ((tot + (_TM - 1)) >> _TMS) << _TMS
            base.append(run_pad + pri)
            end.append(run_pad + pad)
            run_pad = run_pad + pad

        run = [jnp.zeros((16,), jnp.int32)] * _E
        for part, dst in ((0, pos0_v), (1, pos1_v)):
            for v in range(tpw // 16):
                ev = e_v[pl.ds(part * T + t0 + v * 16, 16)]
                posv = jnp.zeros((16,), jnp.int32)
                for ex in range(_E):
                    m = ev == ex
                    inc = _prefix(jnp.where(m, 1, 0))
                    posv = jnp.where(m, base[ex] + run[ex] + inc - 1, posv)
                    run[ex] = run[ex] + _splat(inc, 15)
                dst[pl.ds(v * 16, 16)] = posv
        pltpu.sync_copy(pos0_v, pos_hbm.at[pl.ds(t0, tpw)])
        pltpu.sync_copy(pos1_v, pos_hbm.at[pl.ds(T + t0, tpw)])

        @pl.when(w == 0)
        def _block_map():
            for bv in range(NB_pad // 16):
                b = lax.iota(jnp.int32, 16) + bv * 16
                cnt = jnp.zeros((16,), jnp.int32)
                for ex in range(_E):
                    cnt = cnt + jnp.where(b * _TM >= end[ex], 1, 0)
                be_v[pl.ds(bv * 16, 16)] = jnp.minimum(cnt, _E - 1)
            pltpu.sync_copy(be_v, be_hbm)

        cpx.wait()
        c0 = pltpu.async_copy(rows_v, xp_hbm.at[pos0_v], sem)
        c1 = pltpu.async_copy(rows_v, xp_hbm.at[pos1_v], sem)
        c0.wait()
        c1.wait()

    return permute(ei, x)


def _group_body(be_ref, xp_ref, w1_ref, w2_ref, w3_ref, y_ref):
    xb = xp_ref[...]
    h = _silu(lax.dot_general(xb, w1_ref[0], (((1,), (1,)), ((), ())),
                              preferred_element_type=jnp.float32)) * \
        lax.dot_general(xb, w3_ref[0], (((1,), (1,)), ((), ())),
                        preferred_element_type=jnp.float32)
    y_ref[...] = lax.dot_general(h, w2_ref[0], (((1,), (1,)), ((), ())),
                                 preferred_element_type=jnp.float32)


def _group_matmul(be, x_perm, w1, w2, w3, NB):
    P, D = x_perm.shape
    E, FF, _ = w1.shape
    grid_spec = pltpu.PrefetchScalarGridSpec(
        num_scalar_prefetch=1,
        grid=(NB,),
        in_specs=[
            pl.BlockSpec((_TM, D), lambda b, be_r: (b, 0)),
            pl.BlockSpec((1, FF, D), lambda b, be_r: (be_r[b], 0, 0)),
            pl.BlockSpec((1, D, FF), lambda b, be_r: (be_r[b], 0, 0)),
            pl.BlockSpec((1, FF, D), lambda b, be_r: (be_r[b], 0, 0)),
        ],
        out_specs=pl.BlockSpec((_TM, D), lambda b, be_r: (b, 0)),
    )
    return pl.pallas_call(
        _group_body,
        grid_spec=grid_spec,
        out_shape=jax.ShapeDtypeStruct((NB * _TM, D), jnp.float32),
    )(be, x_perm, w1, w2, w3)


def _combine_call(y, pos, g, T, D):
    tpw = T // _NW
    CT = 16

    @functools.partial(
        pl.kernel, mesh=_sc_mesh(),
        out_type=jax.ShapeDtypeStruct((T, D), jnp.float32),
        scratch_types=[pltpu.VMEM((CT,), jnp.int32),
                       pltpu.VMEM((CT,), jnp.int32),
                       pltpu.VMEM((CT,), jnp.float32),
                       pltpu.VMEM((CT,), jnp.float32),
                       pltpu.VMEM((CT, D), jnp.float32),
                       pltpu.VMEM((CT, D), jnp.float32),
                       pltpu.VMEM((CT, D), jnp.float32),
                       pltpu.SemaphoreType.DMA,
                       pltpu.SemaphoreType.DMA],
    )
    def combine(y_hbm, pos_hbm, g_hbm, out_hbm,
                q0_v, q1_v, g0_v, g1_v, b0_v, b1_v, ob_v, sem0, sem1):
        w = lax.axis_index("s") * _NC + lax.axis_index("c")
        t0 = w * tpw

        def chunk(c, carry):
            tc = t0 + c * CT
            pltpu.sync_copy(pos_hbm.at[pl.ds(tc, CT)], q0_v)
            pltpu.sync_copy(pos_hbm.at[pl.ds(T + tc, CT)], q1_v)
            cp0 = pltpu.async_copy(y_hbm.at[q0_v], b0_v, sem0)
            cp1 = pltpu.async_copy(y_hbm.at[q1_v], b1_v, sem1)
            pltpu.sync_copy(g_hbm.at[pl.ds(tc, CT)], g0_v)
            pltpu.sync_copy(g_hbm.at[pl.ds(T + tc, CT)], g1_v)
            cp0.wait()
            cp1.wait()

            def row(i, carry2):
                iv = jnp.zeros((16,), jnp.int32) + i
                ga = jnp.take(g0_v[...], iv)
                gb = jnp.take(g1_v[...], iv)
                for vv in range(D // 16):
                    sl = pl.ds(vv * 16, 16)
                    ob_v[i, sl] = ga * b0_v[i, sl] + gb * b1_v[i, sl]
                return carry2

            lax.fori_loop(0, CT, row, 0)
            pltpu.sync_copy(ob_v, out_hbm.at[pl.ds(tc, CT), :])
            return carry

        lax.fori_loop(0, tpw // CT, chunk, 0)

    return combine(y, pos, g)


def kernel(x, router_w, w1, w2, w3):
    T, D = x.shape
    E, FF, _ = w1.shape
    A = _K * T
    P = A + E * _TM
    NB = P // _TM
    NB_pad = ((NB + 15) // 16) * 16

    ei, g = _router(x, router_w)
    pos, be, x_perm = _permute_call(ei, x, NB_pad)
    y = _group_matmul(be, x_perm, w1, w2, w3, NB)
    return _combine_call(y, pos, g, T, D)
    pos, be, x_perm = _permute_call(ei, counts, x, NB_pad)
    y = _group_matmul(be, x_perm, w1, w2, w3, NB)
    return _combine_call(y, pos, g, T, D)


# speedup vs baseline: 1.3022x; 1.3022x over previous
import functools

import jax
import jax.numpy as jnp
from jax import lax
from jax.experimental import pallas as pl
from jax.experimental.pallas import tpu as pltpu
from jax.experimental.pallas import tpu_sc as plsc

_E = 8
_K = 2
_TM = 256
_TMS = 8
_NEG = -1e30

_NC = 2
_NS = 16
_NW = _NC * _NS


def _silu(v):
    return v * jax.nn.sigmoid(v)


def _router_body(x_ref, rw_ref, ei_ref, g_ref):
    x = x_ref[...]
    rw = rw_ref[...]
    logits = lax.dot_general(rw, x, (((0,), (1,)), ((), ())),
                             preferred_element_type=jnp.float32)
    row = lax.broadcasted_iota(jnp.int32, logits.shape, 0)
    m1 = jnp.max(logits, axis=0, keepdims=True)
    i1 = jnp.min(jnp.where(logits == m1, row, _E), axis=0, keepdims=True)
    l2 = jnp.where(row == i1, _NEG, logits)
    m2 = jnp.max(l2, axis=0, keepdims=True)
    i2 = jnp.min(jnp.where(l2 == m2, row, _E), axis=0, keepdims=True)
    g1 = jax.nn.sigmoid(m1 - m2)
    T = x.shape[0]
    ei_ref[pl.ds(0, T)] = i1[0]
    ei_ref[pl.ds(T, T)] = i2[0]
    g_ref[pl.ds(0, T)] = g1[0]
    g_ref[pl.ds(T, T)] = 1.0 - g1[0]


def _router(x, router_w):
    T, D = x.shape
    return pl.pallas_call(
        _router_body,
        in_specs=[pl.BlockSpec((T, D), lambda: (0, 0)),
                  pl.BlockSpec((D, _E), lambda: (0, 0))],
        out_specs=[pl.BlockSpec((_K * T,), lambda: (0,)),
                   pl.BlockSpec((_K * T,), lambda: (0,))],
        out_shape=[jax.ShapeDtypeStruct((_K * T,), jnp.int32),
                   jax.ShapeDtypeStruct((_K * T,), jnp.float32)],
    )(x, router_w)


def _sc_mesh():
    return plsc.VectorSubcoreMesh(core_axis_name="c", subcore_axis_name="s")


def _splat(v, j):
    return jnp.take(v, jnp.zeros((16,), jnp.int32) + j)


def _prefix(v):
    lane = lax.iota(jnp.int32, 16)
    for st in (1, 2, 4, 8):
        sh = jnp.take(v, jnp.maximum(lane - st, 0))
        v = v + jnp.where(lane >= st, sh, 0)
    return v


def _permute_call(ei, x, NB_pad):
    T, D = x.shape
    A = _K * T
    tpw = T // _NW
    P = A + _E * _TM
    NV = A // 16

    @functools.partial(
        pl.kernel, mesh=_sc_mesh(),
        out_type=[jax.ShapeDtypeStruct((A,), jnp.int32),
                  jax.ShapeDtypeStruct((NB_pad,), jnp.int32),
                  jax.ShapeDtypeStruct((P, D), jnp.float32)],
        scratch_types=[pltpu.VMEM((A,), jnp.int32),
                       pltpu.VMEM((tpw,), jnp.int32),
                       pltpu.VMEM((tpw,), jnp.int32),
                       pltpu.VMEM((tpw, D), jnp.float32),
                       pltpu.VMEM((NB_pad,), jnp.int32),
                       pltpu.SemaphoreType.DMA],
    )
    def permute(ei_hbm, x_hbm, pos_hbm, be_hbm, xp_hbm,
                e_v, pos0_v, pos1_v, rows_v, be_v, sem):
        w = lax.axis_index("s") * _NC + lax.axis_index("c")
        t0 = w * tpw
        pltpu.sync_copy(ei_hbm, e_v)
        cpx = pltpu.async_copy(x_hbm.at[pl.ds(t0, tpw), :], rows_v, sem)
        wv4 = jnp.zeros((16,), jnp.int32) + w * (tpw // 16)

        def scan(u, carry):
            ev = e_v[pl.ds(u * 16, 16)]
            um = lax.rem(u, NV // _K)
            fac = jnp.clip(wv4 - um, 0, 1)
            out = []
            for ex in range(_E):
                m = jnp.where(ev == ex, 1, 0)
                out.append(carry[2 * ex] + m)
                out.append(carry[2 * ex + 1] + m * fac)
            return tuple(out)

        zero = jnp.zeros((16,), jnp.int32)
        acc = lax.fori_loop(0, NV, scan, (zero,) * (2 * _E))

        run_pad = jnp.zeros((16,), jnp.int32)
        base, end = [], []
        for ex in range(_E):
            tot = _splat(_prefix(acc[2 * ex]), 15)
            pri = _splat(_prefix(acc[2 * ex + 1]), 15)
            pad = ((tot + (_TM - 1)) >> _TMS) << _TMS
            base.append(run_pad + pri)
            end.append(run_pad + pad)
            run_pad = run_pad + pad

        run = [jnp.zeros((16,), jnp.int32)] * _E
        for part, dst in ((0, pos0_v), (1, pos1_v)):
            for v in range(tpw // 16):
                ev = e_v[pl.ds(part * T + t0 + v * 16, 16)]
                posv = jnp.zeros((16,), jnp.int32)
                for ex in range(_E):
                    m = ev == ex
                    inc = _prefix(jnp.where(m, 1, 0))
                    posv = jnp.where(m, base[ex] + run[ex] + inc - 1, posv)
                    run[ex] = run[ex] + _splat(inc, 15)
                dst[pl.ds(v * 16, 16)] = posv
        pltpu.sync_copy(pos0_v, pos_hbm.at[pl.ds(t0, tpw)])
        pltpu.sync_copy(pos1_v, pos_hbm.at[pl.ds(T + t0, tpw)])

        @pl.when(w == 0)
        def _block_map():
            for bv in range(NB_pad // 16):
                b = lax.iota(jnp.int32, 16) + bv * 16
                cnt = jnp.zeros((16,), jnp.int32)
                for ex in range(_E):
                    cnt = cnt + jnp.where(b * _TM >= end[ex], 1, 0)
                be_v[pl.ds(bv * 16, 16)] = jnp.minimum(cnt, _E - 1)
            pltpu.sync_copy(be_v, be_hbm)

        cpx.wait()
        c0 = pltpu.async_copy(rows_v, xp_hbm.at[pos0_v], sem)
        c1 = pltpu.async_copy(rows_v, xp_hbm.at[pos1_v], sem)
        c0.wait()
        c1.wait()

    return permute(ei, x)


def _group_body(be_ref, xp_ref, w1_ref, w2_ref, w3_ref, y_ref):
    xb = xp_ref[...]
    h = _silu(lax.dot_general(xb, w1_ref[0], (((1,), (1,)), ((), ())),
                              preferred_element_type=jnp.float32)) * \
        lax.dot_general(xb, w3_ref[0], (((1,), (1,)), ((), ())),
                        preferred_element_type=jnp.float32)
    y_ref[...] = lax.dot_general(h, w2_ref[0], (((1,), (1,)), ((), ())),
                                 preferred_element_type=jnp.float32)


def _group_matmul(be, x_perm, w1, w2, w3, NB):
    P, D = x_perm.shape
    E, FF, _ = w1.shape
    grid_spec = pltpu.PrefetchScalarGridSpec(
        num_scalar_prefetch=1,
        grid=(NB,),
        in_specs=[
            pl.BlockSpec((_TM, D), lambda b, be_r: (b, 0)),
            pl.BlockSpec((1, FF, D), lambda b, be_r: (be_r[b], 0, 0)),
            pl.BlockSpec((1, D, FF), lambda b, be_r: (be_r[b], 0, 0)),
            pl.BlockSpec((1, FF, D), lambda b, be_r: (be_r[b], 0, 0)),
        ],
        out_specs=pl.BlockSpec((_TM, D), lambda b, be_r: (b, 0)),
    )
    return pl.pallas_call(
        _group_body,
        grid_spec=grid_spec,
        out_shape=jax.ShapeDtypeStruct((NB * _TM, D), jnp.float32),
    )(be, x_perm, w1, w2, w3)


def _combine_call(y, pos, g, T, D):
    tpw = T // _NW
    CT = 16

    @functools.partial(
        pl.kernel, mesh=_sc_mesh(),
        out_type=jax.ShapeDtypeStruct((T, D), jnp.float32),
        scratch_types=[pltpu.VMEM((CT,), jnp.int32),
                       pltpu.VMEM((CT,), jnp.int32),
                       pltpu.VMEM((CT,), jnp.float32),
                       pltpu.VMEM((CT,), jnp.float32),
                       pltpu.VMEM((CT, D), jnp.float32),
                       pltpu.VMEM((CT, D), jnp.float32),
                       pltpu.VMEM((CT, D), jnp.float32),
                       pltpu.SemaphoreType.DMA,
                       pltpu.SemaphoreType.DMA],
    )
    def combine(y_hbm, pos_hbm, g_hbm, out_hbm,
                q0_v, q1_v, g0_v, g1_v, b0_v, b1_v, ob_v, sem0, sem1):
        w = lax.axis_index("s") * _NC + lax.axis_index("c")
        t0 = w * tpw

        def chunk(c, carry):
            tc = t0 + c * CT
            pltpu.sync_copy(pos_hbm.at[pl.ds(tc, CT)], q0_v)
            pltpu.sync_copy(pos_hbm.at[pl.ds(T + tc, CT)], q1_v)
            cp0 = pltpu.async_copy(y_hbm.at[q0_v], b0_v, sem0)
            cp1 = pltpu.async_copy(y_hbm.at[q1_v], b1_v, sem1)
            pltpu.sync_copy(g_hbm.at[pl.ds(tc, CT)], g0_v)
            pltpu.sync_copy(g_hbm.at[pl.ds(T + tc, CT)], g1_v)
            cp0.wait()
            cp1.wait()

            def row(i, carry2):
                iv = jnp.zeros((16,), jnp.int32) + i
                ga = jnp.take(g0_v[...], iv)
                gb = jnp.take(g1_v[...], iv)
                for vv in range(D // 16):
                    sl = pl.ds(vv * 16, 16)
                    ob_v[i, sl] = ga * b0_v[i, sl] + gb * b1_v[i, sl]
                return carry2

            lax.fori_loop(0, CT, row, 0)
            pltpu.sync_copy(ob_v, out_hbm.at[pl.ds(tc, CT), :])
            return carry

        lax.fori_loop(0, tpw // CT, chunk, 0)

    return combine(y, pos, g)


def kernel(x, router_w, w1, w2, w3):
    T, D = x.shape
    E, FF, _ = w1.shape
    A = _K * T
    P = A + E * _TM
    NB = P // _TM
    NB_pad = ((NB + 15) // 16) * 16

    ei, g = _router(x, router_w)
    pos, be, x_perm = _permute_call(ei, x, NB_pad)
    y = _group_matmul(be, x_perm, w1, w2, w3, NB)
    return _combine_call(y, pos, g, T, D)
    pos, be, x_perm = _permute_call(ei, counts, x, NB_pad)
    y = _group_matmul(be, x_perm, w1, w2, w3, NB)
    return _combine_call(y, pos, g, T, D)
